# baseline (speedup 1.0000x reference)
import functools

import jax
import jax.numpy as jnp
from jax import lax
from jax.experimental import pallas as pl
from jax.experimental.pallas import tpu as pltpu
from jax.experimental.pallas import tpu_sc as plsc

B = 16384
V_SMALL = 1000
V_DRUG = 100000
D_EMB = 32
D_DRUG = 64
D_LIN = 32

NC = 2
NS = 16
NW = NC * NS
CH = 4096
NCHK = B // CH
NIB = 2
UNROLL = 8
GROUPS = CH // 16


def _zero_entry0(row_v):
    m = jnp.where(lax.iota(jnp.int32, 16) == 0, 0.0, 1.0)
    row_v[pl.ds(0, 16)] = row_v[pl.ds(0, 16)] * m


def _gather_chunk2(row_v, idx_v, out_v, obase):
    def body(g, carry):
        base = g * (16 * UNROLL)
        for u in range(UNROLL):
            off = base + u * 16
            idx16 = idx_v[pl.ds(off, 16)]
            out_v[pl.ds(obase + off, 16)] = plsc.load_gather(row_v, [idx16])
        return carry

    lax.fori_loop(0, GROUPS // UNROLL, body, 0)


def _make_gather4():
    mesh = plsc.VectorSubcoreMesh(core_axis_name="c", subcore_axis_name="s",
                                  num_cores=NC, num_subcores=NS)

    @functools.partial(
        pl.kernel,
        mesh=mesh,
        compiler_params=pltpu.CompilerParams(needs_layout_passes=False),
        out_type=(
            jax.ShapeDtypeStruct((D_EMB, B), jnp.float32),
            jax.ShapeDtypeStruct((D_DRUG, B), jnp.float32),
            jax.ShapeDtypeStruct((D_EMB, B), jnp.float32),
            jax.ShapeDtypeStruct((D_EMB, B), jnp.float32),
        ),
        scratch_types=[
            pltpu.VMEM((V_SMALL,), jnp.float32),
            pltpu.VMEM((V_DRUG,), jnp.float32),
            pltpu.VMEM((V_SMALL,), jnp.float32),
            pltpu.VMEM((V_SMALL,), jnp.float32),
            pltpu.VMEM((CH,), jnp.int32),
            pltpu.VMEM((CH,), jnp.int32),
            pltpu.VMEM((B,), jnp.float32),
            pltpu.SemaphoreType.DMA,
            pltpu.SemaphoreType.DMA,
            pltpu.SemaphoreType.DMA,
        ],
    )
    def gather4(mt_h, dn_h, cs_h, ns_h, wmt_h, wdt_h, wct_h, wnt_h,
                omt_h, odt_h, oct_h, ont_h,
                row_m, row_d, row_c, row_n,
                ib0, ib1, ob_full,
                sem_row, sem_in, sem_out):
        w = lax.axis_index("s") * NC + lax.axis_index("c")
        ibufs = (ib0, ib1)

        rm = pltpu.async_copy(wmt_h.at[w], row_m, sem_row)
        rd = pltpu.async_copy(wdt_h.at[w], row_d, sem_row)
        rc = pltpu.async_copy(wct_h.at[w], row_c, sem_row)
        rn = pltpu.async_copy(wnt_h.at[w], row_n, sem_row)

        passes = [
            (row_m, rm, mt_h, omt_h, w),
            (row_d, rd, dn_h, odt_h, w),
            (row_c, rc, cs_h, oct_h, w),
            (row_n, rn, ns_h, ont_h, w),
            (row_d, None, dn_h, odt_h, w + NW),
        ]
        steps = [(p, ck) for p in passes for ck in range(NCHK)]
        n = len(steps)

        pulls = [None] * n
        writes = [None] * n

        def pull(i):
            (_, _, idx_h, _, _), ck = steps[i]
            pulls[i] = pltpu.async_copy(
                idx_h.at[pl.ds(ck * CH, CH)], ibufs[i % NIB], sem_in)

        pull(0)
        rd2 = None
        prev_write = None
        for i in range(n):
            (row_v, rcopy, _, out_h, orow), ck = steps[i]
            if i + 1 < n:
                pull(i + 1)
            if ck == 0:
                if rcopy is not None:
                    rcopy.wait()
                else:
                    rd2.wait()
                _zero_entry0(row_v)
                if prev_write is not None:
                    prev_write.wait()
            pulls[i].wait()
            _gather_chunk2(row_v, ibufs[i % NIB], ob_full, ck * CH)
            if row_v is row_d and ck == NCHK - 1 and rd2 is None:
                rd2 = pltpu.async_copy(wdt_h.at[w + NW], row_d, sem_row)
            if ck == NCHK - 1:
                prev_write = pltpu.async_copy(ob_full, out_h.at[orow],
                                              sem_out)
        prev_write.wait()

    return gather4


_gather4 = _make_gather4()


LIN_BLK = 2048


def _lin_body(x1, x2, x3, x4, x5, w_ref, b_ref, o1, o2, o3, o4, o5):
    for k, (x, o) in enumerate(((x1, o1), (x2, o2), (x3, o3), (x4, o4),
                                (x5, o5))):
        o[...] = w_ref[k] * x[...][None, :] + b_ref[k]


def _lin5(xs, ws, bs):
    x_spec = pl.BlockSpec((LIN_BLK,), lambda i: (i,))
    wb_spec = pl.BlockSpec((5, D_LIN, 1), lambda i: (0, 0, 0))
    o_spec = pl.BlockSpec((D_LIN, LIN_BLK), lambda i: (0, i))
    w5 = jnp.stack([w.reshape(D_LIN) for w in ws])[:, :, None]
    b5 = jnp.stack([b.reshape(D_LIN) for b in bs])[:, :, None]
    outs = pl.pallas_call(
        _lin_body,
        grid=(B // LIN_BLK,),
        in_specs=[x_spec] * 5 + [wb_spec, wb_spec],
        out_specs=[o_spec] * 5,
        out_shape=[jax.ShapeDtypeStruct((D_LIN, B), jnp.float32)] * 5,
    )(*[x.reshape(B) for x in xs], w5, b5)
    return [o.T for o in outs]


def kernel(media_type, temperature, pre_culture_time, pre_culture_od600,
           drug_culture_time, drug_name, concentration, carbon_source,
           nitrogen_source, W_media, W_drug, W_carbon, W_nitrogen,
           W_temp, b_temp, W_pct, b_pct, W_od, b_od, W_dct, b_dct,
           W_conc, b_conc):
    mt = media_type.astype(jnp.int32)
    dn = drug_name.astype(jnp.int32)
    cs = carbon_source.astype(jnp.int32)
    ns_ = nitrogen_source.astype(jnp.int32)

    omt, odt, oct_, ont = _gather4(
        mt, dn, cs, ns_, W_media.T, W_drug.T, W_carbon.T, W_nitrogen.T)

    lt, lpct, lod, ldct, lconc = _lin5(
        (temperature, pre_culture_time, pre_culture_od600, drug_culture_time,
         concentration),
        (W_temp, W_pct, W_od, W_dct, W_conc),
        (b_temp, b_pct, b_od, b_dct, b_conc))

    return (omt.T, lt, lpct, lod, ldct, odt.T, lconc, oct_.T, ont.T)


# speedup vs baseline: 1.0509x; 1.0509x over previous
import functools

import jax
import jax.numpy as jnp
from jax import lax
from jax.experimental import pallas as pl
from jax.experimental.pallas import tpu as pltpu
from jax.experimental.pallas import tpu_sc as plsc

B = 16384
V_SMALL = 1000
V_DRUG = 100000
D_EMB = 32
D_DRUG = 64
D_LIN = 32

NC = 2
NS = 16
NW = NC * NS
CH = 4096
NCHK = B // CH
NBUF = 3
UNROLL = 8
GROUPS = CH // 16


def _zero_entry0(row_v):
    m = jnp.where(lax.iota(jnp.int32, 16) == 0, 0.0, 1.0)
    row_v[pl.ds(0, 16)] = row_v[pl.ds(0, 16)] * m


def _gather_chunk(row_v, idx_v, out_v):
    def body(g, carry):
        base = g * (16 * UNROLL)
        for u in range(UNROLL):
            off = base + u * 16
            idx16 = idx_v[pl.ds(off, 16)]
            out_v[pl.ds(off, 16)] = plsc.load_gather(row_v, [idx16])
        return carry

    lax.fori_loop(0, GROUPS // UNROLL, body, 0)


def _make_gather4():
    mesh = plsc.VectorSubcoreMesh(core_axis_name="c", subcore_axis_name="s",
                                  num_cores=NC, num_subcores=NS)

    @functools.partial(
        pl.kernel,
        mesh=mesh,
        compiler_params=pltpu.CompilerParams(needs_layout_passes=False),
        out_type=(
            jax.ShapeDtypeStruct((D_EMB, B), jnp.float32),
            jax.ShapeDtypeStruct((D_DRUG, B), jnp.float32),
            jax.ShapeDtypeStruct((D_EMB, B), jnp.float32),
            jax.ShapeDtypeStruct((D_EMB, B), jnp.float32),
        ),
        scratch_types=[
            pltpu.VMEM((V_SMALL,), jnp.float32),
            pltpu.VMEM((V_DRUG,), jnp.float32),
            pltpu.VMEM((V_SMALL,), jnp.float32),
            pltpu.VMEM((V_SMALL,), jnp.float32),
            pltpu.VMEM((CH,), jnp.int32),
            pltpu.VMEM((CH,), jnp.int32),
            pltpu.VMEM((CH,), jnp.int32),
            pltpu.VMEM((CH,), jnp.float32),
            pltpu.VMEM((CH,), jnp.float32),
            pltpu.VMEM((CH,), jnp.float32),
            pltpu.SemaphoreType.DMA,
            pltpu.SemaphoreType.DMA,
            pltpu.SemaphoreType.DMA,
        ],
    )
    def gather4(mt_h, dn_h, cs_h, ns_h, wmt_h, wdt_h, wct_h, wnt_h,
                omt_h, odt_h, oct_h, ont_h,
                row_m, row_d, row_c, row_n,
                ib0, ib1, ib2, ob0, ob1, ob2,
                sem_row, sem_in, sem_out):
        w = lax.axis_index("s") * NC + lax.axis_index("c")
        ibufs = (ib0, ib1, ib2)
        obufs = (ob0, ob1, ob2)

        rm = pltpu.async_copy(wmt_h.at[w], row_m, sem_row)
        rd = pltpu.async_copy(wdt_h.at[w], row_d, sem_row)
        rc = pltpu.async_copy(wct_h.at[w], row_c, sem_row)
        rn = pltpu.async_copy(wnt_h.at[w], row_n, sem_row)

        passes = [
            (row_m, rm, mt_h, omt_h, w),
            (row_d, rd, dn_h, odt_h, w),
            (row_c, rc, cs_h, oct_h, w),
            (row_n, rn, ns_h, ont_h, w),
            (row_d, None, dn_h, odt_h, w + NW),
        ]
        steps = [(p, ck) for p in passes for ck in range(NCHK)]
        n = len(steps)

        pulls = [None] * n
        writes = [None] * n

        def pull(i):
            (_, _, idx_h, _, _), ck = steps[i]
            pulls[i] = pltpu.async_copy(
                idx_h.at[pl.ds(ck * CH, CH)], ibufs[i % NBUF], sem_in)

        pull(0)
        pull(1)
        rd2 = None
        for i in range(n):
            (row_v, rcopy, _, out_h, orow), ck = steps[i]
            if i + 2 < n:
                pull(i + 2)
            if ck == 0:
                if rcopy is not None:
                    rcopy.wait()
                else:
                    rd2.wait()
                _zero_entry0(row_v)
            pulls[i].wait()
            if i >= NBUF:
                writes[i - NBUF].wait()
            _gather_chunk(row_v, ibufs[i % NBUF], obufs[i % NBUF])
            if row_v is row_d and ck == NCHK - 1 and rd2 is None:
                rd2 = pltpu.async_copy(wdt_h.at[w + NW], row_d, sem_row)
            writes[i] = pltpu.async_copy(
                obufs[i % NBUF], out_h.at[orow, pl.ds(ck * CH, CH)], sem_out)
        for i in range(n - NBUF, n):
            writes[i].wait()

    return gather4


_gather4 = _make_gather4()


LIN_BLK = 2048


def _lin_body(x1, x2, x3, x4, x5, w_ref, b_ref, o1, o2, o3, o4, o5):
    for k, (x, o) in enumerate(((x1, o1), (x2, o2), (x3, o3), (x4, o4),
                                (x5, o5))):
        o[...] = w_ref[k] * x[...][None, :] + b_ref[k]


def _lin5(xs, ws, bs):
    x_spec = pl.BlockSpec((LIN_BLK,), lambda i: (i,))
    wb_spec = pl.BlockSpec((5, D_LIN, 1), lambda i: (0, 0, 0))
    o_spec = pl.BlockSpec((D_LIN, LIN_BLK), lambda i: (0, i))
    w5 = jnp.stack([w.reshape(D_LIN) for w in ws])[:, :, None]
    b5 = jnp.stack([b.reshape(D_LIN) for b in bs])[:, :, None]
    outs = pl.pallas_call(
        _lin_body,
        grid=(B // LIN_BLK,),
        in_specs=[x_spec] * 5 + [wb_spec, wb_spec],
        out_specs=[o_spec] * 5,
        out_shape=[jax.ShapeDtypeStruct((D_LIN, B), jnp.float32)] * 5,
    )(*[x.reshape(B) for x in xs], w5, b5)
    return [o.T for o in outs]


def kernel(media_type, temperature, pre_culture_time, pre_culture_od600,
           drug_culture_time, drug_name, concentration, carbon_source,
           nitrogen_source, W_media, W_drug, W_carbon, W_nitrogen,
           W_temp, b_temp, W_pct, b_pct, W_od, b_od, W_dct, b_dct,
           W_conc, b_conc):
    mt = media_type.astype(jnp.int32)
    dn = drug_name.astype(jnp.int32)
    cs = carbon_source.astype(jnp.int32)
    ns_ = nitrogen_source.astype(jnp.int32)

    omt, odt, oct_, ont = _gather4(
        mt, dn, cs, ns_, W_media.T, W_drug.T, W_carbon.T, W_nitrogen.T)

    lt, lpct, lod, ldct, lconc = _lin5(
        (temperature, pre_culture_time, pre_culture_od600, drug_culture_time,
         concentration),
        (W_temp, W_pct, W_od, W_dct, W_conc),
        (b_temp, b_pct, b_od, b_dct, b_conc))

    return (omt.T, lt, lpct, lod, ldct, odt.T, lconc, oct_.T, ont.T)
